# Initial kernel scaffold; baseline (speedup 1.0000x reference)
import jax
import jax.numpy as jnp
from jax.experimental import pallas as pl


def kernel(x, table):
    raise NotImplementedError("write your pallas kernel here")


# speedup vs baseline: 2.3437x; 2.3437x over previous
import dataclasses
import functools

import jax
import jax.numpy as jnp
import numpy as np
from jax.experimental import pallas as pl
from jax.experimental.pallas import tpu as pltpu
from jax.experimental.pallas import tpu_sc as plsc

_W = 128
_L = 16


@functools.cache
def _make_lookup(B, V, D, dtype, scale):
    mesh = plsc.VectorSubcoreMesh(core_axis_name="core", subcore_axis_name="subcore")
    cp = pltpu.CompilerParams()
    if "needs_layout_passes" in pltpu.CompilerParams.__dataclass_fields__:
        cp = dataclasses.replace(cp, needs_layout_passes=False)

    @functools.partial(
        pl.kernel,
        out_type=jax.ShapeDtypeStruct((B * D,), dtype),
        mesh=mesh,
        scratch_types=[pltpu.VMEM((V * D,), dtype)],
        compiler_params=cp,
    )
    def lookup(table_hbm, idx_hbm, out_hbm, tab_vmem):
        pltpu.sync_copy(table_hbm, tab_vmem)

        @pl.loop(0, V * D // _L)
        def _(i):
            sl = pl.ds(i * _L, _L)
            tab_vmem[sl] = tab_vmem[sl] * scale

        lanes_d = jax.lax.iota(jnp.int32, _L) * D

        def body(i_vmem, o_vmem):
            for g in range(_W // _L):
                idxv = i_vmem[0, pl.ds(g * _L, _L)]
                src = idxv * D
                dst = lanes_d + (g * _L * D)
                for j in range(D):
                    vals = plsc.load_gather(tab_vmem, [src + j])
                    plsc.store_scatter(o_vmem, [dst + j], vals)

        pltpu.emit_pipeline(
            body,
            grid=(B // _W,),
            in_specs=[pl.BlockSpec((1, _W), index_map=lambda i: (0, i))],
            out_specs=[pl.BlockSpec((_W * D,), index_map=lambda i: (i,))],
            core_axis_name=("core", "subcore"),
            dimension_semantics=(pltpu.PARALLEL,),
        )(idx_hbm, out_hbm)

    return lookup


def kernel(x, table):
    V, D = table.shape
    B = x.size
    scale = float(np.sqrt(D).astype(np.float32))
    idx = x.reshape(1, B)
    out = _make_lookup(B, V, D, table.dtype, scale)(table.reshape(V * D), idx)
    return out.reshape(*x.shape, D)
